# baseline (speedup 1.0000x reference)
import functools

import numpy as np
import jax
import jax.numpy as jnp
from jax import lax
from jax.experimental import pallas as pl
from jax.experimental.pallas import tpu as pltpu
from jax.experimental.pallas import tpu_sc as plsc

def _pad_rows_idx(idx, mult):
    r = (-idx.shape[0]) % mult
    return jnp.pad(idx, (0, r)) if r else idx


_NW = 32
_P = 112
_BLK = _NW * _P


def _sc_gather(table, idx):
    B = int(idx.shape[0])
    D = int(table.shape[1])
    ppw = B // _BLK
    nbuf = min(ppw, 3 if D > 128 else 4)
    mesh = plsc.VectorSubcoreMesh(core_axis_name="c", subcore_axis_name="s")

    def body(table_hbm, idx_hbm, out_hbm, idx_v, rows_v, gsem, osem):
        wid = lax.axis_index("s") * 2 + lax.axis_index("c")
        base = wid * (ppw * _P)
        pltpu.sync_copy(idx_hbm.at[pl.ds(base, ppw * _P)], idx_v)

        def gcopy(s, i):
            return pltpu.make_async_copy(
                table_hbm.at[idx_v.at[pl.ds(i * _P, _P)]],
                rows_v.at[s], gsem.at[s])

        def ocopy(s, i):
            return pltpu.make_async_copy(
                rows_v.at[s], out_hbm.at[pl.ds(base + i * _P, _P)],
                osem.at[s])

        for s in range(nbuf):
            gcopy(s, s).start()
        for i in range(ppw):
            s = i % nbuf
            gcopy(s, i).wait()
            ocopy(s, i).start()
            if i + nbuf < ppw:
                ocopy(s, i).wait()
                gcopy(s, i + nbuf).start()
        for i in range(max(ppw - nbuf, 0), ppw):
            ocopy(i % nbuf, i).wait()

    k = pl.kernel(
        body,
        out_type=jax.ShapeDtypeStruct((B, D), jnp.float32),
        mesh=mesh,
        scratch_types=[
            pltpu.VMEM((ppw * _P,), jnp.int32),
            pltpu.VMEM((nbuf, _P, D), jnp.float32),
            pltpu.SemaphoreType.DMA((nbuf,)),
            pltpu.SemaphoreType.DMA((nbuf,)),
        ],
        compiler_params=pltpu.CompilerParams(use_tc_tiling_on_sc=False),
    )
    return k(table, idx)


def _dot_t(a, w):
    return lax.dot_general(
        a, w, (((1,), (1,)), ((), ())),
        preferred_element_type=jnp.float32,
        precision=lax.Precision.DEFAULT,
    )


def _tc_conv(G, Wm, b, g, bt, n, Wu=None, bu=None):
    R = int(G.shape[0])
    fo = int(Wm.shape[0])
    act = g is not None
    up = Wu is not None
    fu = int(Wu.shape[0]) if up else fo

    def body(*refs):
        if act and up:
            G_ref, W_ref, b_ref, g_ref, bt_ref, Wu_ref, bu_ref, o_ref, y_ref = refs
        elif act:
            G_ref, W_ref, b_ref, g_ref, bt_ref, o_ref, y_ref = refs
        else:
            G_ref, W_ref, b_ref, o_ref, y_ref = refs
        y_ref[...] = _dot_t(G_ref[...], W_ref[...]) + b_ref[...]
        if act:
            mask = (lax.broadcasted_iota(jnp.int32, (R, 1), 0) < n
                    ).astype(jnp.float32)
            mu = jnp.sum(y_ref[...] * mask, 0, keepdims=True) * (1.0 / n)
            yc = y_ref[...] - mu
            var = jnp.sum(yc * yc * mask, 0, keepdims=True) * (1.0 / n)
            y_ref[...] = jnp.tanh(
                (y_ref[...] - mu) * lax.rsqrt(var + 1e-5) * g_ref[...]
                + bt_ref[...])
        if up:
            o_ref[...] = _dot_t(y_ref[...], Wu_ref[...]) + bu_ref[...]
        else:
            o_ref[...] = y_ref[...]

    ins = [G, Wm, b.reshape(1, fo)]
    if act:
        ins += [g.reshape(1, fo), bt.reshape(1, fo)]
    if up:
        ins += [Wu, bu.reshape(1, fu)]
    return pl.pallas_call(
        body,
        out_shape=jax.ShapeDtypeStruct((R, fu), jnp.float32),
        scratch_shapes=[pltpu.VMEM((R, fo), jnp.float32)],
    )(*ins)


def _tc_matmul(A, M):
    R = int(A.shape[0])
    f = int(M.shape[1])

    def body(A_ref, M_ref, o_ref):
        o_ref[...] = lax.dot_general(
            A_ref[...], M_ref[...], (((1,), (0,)), ((), ())),
            preferred_element_type=jnp.float32,
            precision=lax.Precision.DEFAULT,
        )

    return pl.pallas_call(
        body,
        out_shape=jax.ShapeDtypeStruct((R, f), jnp.float32),
    )(A, M)


def _pool_mat(f):
    S = np.zeros((7 * f, f), np.float32)
    S[np.arange(7 * f), np.repeat(np.arange(f), 7)] = 1.0 / 7.0
    return jnp.asarray(S)


def _pair_mat(f):
    P = np.zeros((2 * f, f), np.float32)
    P[np.arange(2 * f), np.repeat(np.arange(f), 2)] = 0.5
    return jnp.asarray(P)


def kernel(x, params, neigh_10242, neigh_2562, neigh_642, neigh_162,
           up_top_642, up_down_642, up_top_2562, up_down_2562,
           up_top_10242, up_down_10242):
    p = params

    x16 = jnp.pad(x, ((0, 0), (0, 12)))
    W11 = p['c1_1_W'].reshape(32, 7, 4)
    W11 = jnp.pad(W11, ((0, 0), (0, 0), (0, 12))).reshape(32, 112)

    i1 = _pad_rows_idx(neigh_10242, _BLK)
    i2 = _pad_rows_idx(neigh_2562, _BLK)
    i3 = _pad_rows_idx(neigh_642, _BLK)
    i4 = _pad_rows_idx(neigh_162, _BLK)

    def conv(h, idx, n, name, Wm=None, up=None):
        D = int(h.shape[1])
        G = _sc_gather(h, idx).reshape(-1, 7 * D)
        Wm = p[name + '_W'] if Wm is None else Wm
        if name == 'c10':
            return _tc_conv(G, Wm, p[name + '_b'], None, None, n)
        Wu, bu = (p[up + '_W'], p[up + '_b']) if up else (None, None)
        return _tc_conv(G, Wm, p[name + '_b'], p[name + '_g'],
                        p[name + '_bt'], n, Wu, bu)

    def pool(h, neigh_full, num):
        D = int(h.shape[1])
        idx = _pad_rows_idx(neigh_full[:num * 7], _BLK)
        G = _sc_gather(h, idx).reshape(-1, 7 * D)
        return _tc_matmul(G, _pool_mat(D))

    def up_finish(y, down, raw, num, fo, skip):
        tab = y.reshape(-1, fo)
        x1 = y[:raw, :fo]
        Z = _sc_gather(tab, _pad_rows_idx(down, _BLK)).reshape(-1, 2 * fo)
        x2 = _tc_matmul(Z, _pair_mat(fo))[:num - raw]
        return jnp.concatenate([jnp.concatenate([x1, x2], 0), skip[:num]], 1)

    h = conv(x16, i1, 10242, 'c1_1', Wm=W11)
    x1s = conv(h, i1, 10242, 'c1_2')
    h = pool(x1s, neigh_10242, 2562)
    h = conv(h, i2, 2562, 'c2_1')
    x2s = conv(h, i2, 2562, 'c2_2')
    h = pool(x2s, neigh_2562, 642)
    h = conv(h, i3, 642, 'c3_1')
    x3s = conv(h, i3, 642, 'c3_2')
    h = pool(x3s, neigh_642, 162)
    h = conv(h, i4, 162, 'c4_1')
    y1 = conv(h, i4, 162, 'c4_2', up='u1')
    h = up_finish(y1, up_down_642, 162, 642, 128, x3s)
    h = conv(h, i3, 642, 'c7_1')
    y2 = conv(h, i3, 642, 'c7_2', up='u2')
    h = up_finish(y2, up_down_2562, 642, 2562, 64, x2s)
    h = conv(h, i2, 2562, 'c8_1')
    y3 = conv(h, i2, 2562, 'c8_2', up='u3')
    h = up_finish(y3, up_down_10242, 2562, 10242, 32, x1s)
    h = conv(h, i1, 10242, 'c9_1')
    h = conv(h, i1, 10242, 'c9_2')
    return conv(h, i1, 10242, 'c10')[:10242]


# speedup vs baseline: 1.0030x; 1.0030x over previous
import functools

import numpy as np
import jax
import jax.numpy as jnp
from jax import lax
from jax.experimental import pallas as pl
from jax.experimental.pallas import tpu as pltpu
from jax.experimental.pallas import tpu_sc as plsc

def _pad_rows_idx(idx, mult):
    r = (-idx.shape[0]) % mult
    return jnp.pad(idx, (0, r)) if r else idx


_NW = 32
_P = 112
_BLK = _NW * _P


def _sc_gather(table, idx):
    B = int(idx.shape[0])
    D = int(table.shape[1])
    ppw = B // _BLK
    wmax = max(1, min(ppw, (430 * 1024) // (_P * D * 4)))
    nwaves = -(-ppw // wmax)
    mesh = plsc.VectorSubcoreMesh(core_axis_name="c", subcore_axis_name="s")

    def body(table_hbm, idx_hbm, out_hbm, idx_v, rows_v, gsem, osem):
        wid = lax.axis_index("s") * 2 + lax.axis_index("c")
        base = wid * (ppw * _P)
        pltpu.sync_copy(idx_hbm.at[pl.ds(base, ppw * _P)], idx_v)

        def gcopy(i, slot):
            return pltpu.make_async_copy(
                table_hbm.at[idx_v.at[pl.ds(i * _P, _P)]],
                rows_v.at[pl.ds(slot * _P, _P)], gsem)

        for w in range(nwaves):
            lo = w * wmax
            k = min(wmax, ppw - lo)
            for s in range(k):
                gcopy(lo + s, s).start()
            for s in range(k):
                gcopy(lo + s, s).wait()
            ocp = pltpu.make_async_copy(
                rows_v.at[pl.ds(0, k * _P)],
                out_hbm.at[pl.ds(base + lo * _P, k * _P)], osem)
            ocp.start()
            ocp.wait()

    k = pl.kernel(
        body,
        out_type=jax.ShapeDtypeStruct((B, D), jnp.float32),
        mesh=mesh,
        scratch_types=[
            pltpu.VMEM((ppw * _P,), jnp.int32),
            pltpu.VMEM((min(ppw, wmax) * _P, D), jnp.float32),
            pltpu.SemaphoreType.DMA,
            pltpu.SemaphoreType.DMA,
        ],
        compiler_params=pltpu.CompilerParams(use_tc_tiling_on_sc=False),
    )
    return k(table, idx)


def _dot_t(a, w):
    return lax.dot_general(
        a, w, (((1,), (1,)), ((), ())),
        preferred_element_type=jnp.float32,
        precision=lax.Precision.DEFAULT,
    )


def _tc_conv(G, Wm, b, g, bt, n, Wu=None, bu=None):
    R = int(G.shape[0])
    fo = int(Wm.shape[0])
    act = g is not None
    up = Wu is not None
    fu = int(Wu.shape[0]) if up else fo

    def body(*refs):
        if act and up:
            G_ref, W_ref, b_ref, g_ref, bt_ref, Wu_ref, bu_ref, o_ref, y_ref = refs
        elif act:
            G_ref, W_ref, b_ref, g_ref, bt_ref, o_ref, y_ref = refs
        else:
            G_ref, W_ref, b_ref, o_ref, y_ref = refs
        y_ref[...] = _dot_t(G_ref[...], W_ref[...]) + b_ref[...]
        if act:
            mask = (lax.broadcasted_iota(jnp.int32, (R, 1), 0) < n
                    ).astype(jnp.float32)
            mu = jnp.sum(y_ref[...] * mask, 0, keepdims=True) * (1.0 / n)
            yc = y_ref[...] - mu
            var = jnp.sum(yc * yc * mask, 0, keepdims=True) * (1.0 / n)
            y_ref[...] = jnp.tanh(
                (y_ref[...] - mu) * lax.rsqrt(var + 1e-5) * g_ref[...]
                + bt_ref[...])
        if up:
            o_ref[...] = _dot_t(y_ref[...], Wu_ref[...]) + bu_ref[...]
        else:
            o_ref[...] = y_ref[...]

    ins = [G, Wm, b.reshape(1, fo)]
    if act:
        ins += [g.reshape(1, fo), bt.reshape(1, fo)]
    if up:
        ins += [Wu, bu.reshape(1, fu)]
    return pl.pallas_call(
        body,
        out_shape=jax.ShapeDtypeStruct((R, fu), jnp.float32),
        scratch_shapes=[pltpu.VMEM((R, fo), jnp.float32)],
    )(*ins)


def _tc_matmul(A, M):
    R = int(A.shape[0])
    f = int(M.shape[1])

    def body(A_ref, M_ref, o_ref):
        o_ref[...] = lax.dot_general(
            A_ref[...], M_ref[...], (((1,), (0,)), ((), ())),
            preferred_element_type=jnp.float32,
            precision=lax.Precision.DEFAULT,
        )

    return pl.pallas_call(
        body,
        out_shape=jax.ShapeDtypeStruct((R, f), jnp.float32),
    )(A, M)


def _pool_mat(f):
    S = np.zeros((7 * f, f), np.float32)
    S[np.arange(7 * f), np.repeat(np.arange(f), 7)] = 1.0 / 7.0
    return jnp.asarray(S)


def _pair_mat(f):
    P = np.zeros((2 * f, f), np.float32)
    P[np.arange(2 * f), np.repeat(np.arange(f), 2)] = 0.5
    return jnp.asarray(P)


def kernel(x, params, neigh_10242, neigh_2562, neigh_642, neigh_162,
           up_top_642, up_down_642, up_top_2562, up_down_2562,
           up_top_10242, up_down_10242):
    p = params

    x16 = jnp.pad(x, ((0, 0), (0, 12)))
    W11 = p['c1_1_W'].reshape(32, 7, 4)
    W11 = jnp.pad(W11, ((0, 0), (0, 0), (0, 12))).reshape(32, 112)

    i1 = _pad_rows_idx(neigh_10242, _BLK)
    i2 = _pad_rows_idx(neigh_2562, _BLK)
    i3 = _pad_rows_idx(neigh_642, _BLK)
    i4 = _pad_rows_idx(neigh_162, _BLK)

    def conv(h, idx, n, name, Wm=None, up=None):
        D = int(h.shape[1])
        G = _sc_gather(h, idx).reshape(-1, 7 * D)
        Wm = p[name + '_W'] if Wm is None else Wm
        if name == 'c10':
            return _tc_conv(G, Wm, p[name + '_b'], None, None, n)
        Wu, bu = (p[up + '_W'], p[up + '_b']) if up else (None, None)
        return _tc_conv(G, Wm, p[name + '_b'], p[name + '_g'],
                        p[name + '_bt'], n, Wu, bu)

    def pool(h, neigh_full, num):
        D = int(h.shape[1])
        idx = _pad_rows_idx(neigh_full[:num * 7], _BLK)
        G = _sc_gather(h, idx).reshape(-1, 7 * D)
        return _tc_matmul(G, _pool_mat(D))

    def up_finish(y, down, raw, num, fo, skip):
        tab = y.reshape(-1, fo)
        x1 = y[:raw, :fo]
        Z = _sc_gather(tab, _pad_rows_idx(down, _BLK)).reshape(-1, 2 * fo)
        x2 = _tc_matmul(Z, _pair_mat(fo))[:num - raw]
        return jnp.concatenate([jnp.concatenate([x1, x2], 0), skip[:num]], 1)

    h = conv(x16, i1, 10242, 'c1_1', Wm=W11)
    x1s = conv(h, i1, 10242, 'c1_2')
    h = pool(x1s, neigh_10242, 2562)
    h = conv(h, i2, 2562, 'c2_1')
    x2s = conv(h, i2, 2562, 'c2_2')
    h = pool(x2s, neigh_2562, 642)
    h = conv(h, i3, 642, 'c3_1')
    x3s = conv(h, i3, 642, 'c3_2')
    h = pool(x3s, neigh_642, 162)
    h = conv(h, i4, 162, 'c4_1')
    y1 = conv(h, i4, 162, 'c4_2', up='u1')
    h = up_finish(y1, up_down_642, 162, 642, 128, x3s)
    h = conv(h, i3, 642, 'c7_1')
    y2 = conv(h, i3, 642, 'c7_2', up='u2')
    h = up_finish(y2, up_down_2562, 642, 2562, 64, x2s)
    h = conv(h, i2, 2562, 'c8_1')
    y3 = conv(h, i2, 2562, 'c8_2', up='u3')
    h = up_finish(y3, up_down_10242, 2562, 10242, 32, x1s)
    h = conv(h, i1, 10242, 'c9_1')
    h = conv(h, i1, 10242, 'c9_2')
    return conv(h, i1, 10242, 'c10')[:10242]


# speedup vs baseline: 3.9452x; 3.9336x over previous
import functools

import numpy as np
import jax
import jax.numpy as jnp
from jax import lax
from jax.experimental import pallas as pl
from jax.experimental.pallas import tpu as pltpu
from jax.experimental.pallas import tpu_sc as plsc

def _pad_rows_idx(idx, mult):
    r = (-idx.shape[0]) % mult
    return jnp.pad(idx, (0, r)) if r else idx


_W = 128
_BLK = 7 * _W


def _sc_gather(table, idx):
    B = int(idx.shape[0])
    V = int(table.shape[0])
    D = int(table.shape[1])
    mesh = plsc.VectorSubcoreMesh(core_axis_name="c", subcore_axis_name="s")

    def body(table_hbm, idx_hbm, out_hbm, tab_sh):
        @pl.when(lax.axis_index("s") == 0)
        def _():
            pltpu.sync_copy(table_hbm, tab_sh)
        plsc.subcore_barrier()

        def inner(i_vmem, o_vmem):
            pltpu.sync_copy(tab_sh.at[i_vmem.at[0]], o_vmem)

        pltpu.emit_pipeline(
            inner,
            grid=(B // _W,),
            in_specs=[pl.BlockSpec((1, _W), index_map=lambda i: (0, i))],
            out_specs=[pl.BlockSpec((_W, D), index_map=lambda i: (i, 0))],
            core_axis_name=("c", "s"),
            dimension_semantics=(pltpu.PARALLEL,),
        )(idx_hbm, out_hbm)

    k = pl.kernel(
        body,
        out_type=jax.ShapeDtypeStruct((B, D), jnp.float32),
        mesh=mesh,
        scratch_types=[pltpu.VMEM_SHARED((V, D), jnp.float32)],
        compiler_params=pltpu.CompilerParams(use_tc_tiling_on_sc=False),
    )
    return k(table, idx.reshape(1, B))


def _dot_t(a, w):
    return lax.dot_general(
        a, w, (((1,), (1,)), ((), ())),
        preferred_element_type=jnp.float32,
        precision=lax.Precision.DEFAULT,
    )


def _tc_conv(G, Wm, b, g, bt, n, Wu=None, bu=None):
    R = int(G.shape[0])
    fo = int(Wm.shape[0])
    act = g is not None
    up = Wu is not None
    fu = int(Wu.shape[0]) if up else fo

    def body(*refs):
        if act and up:
            G_ref, W_ref, b_ref, g_ref, bt_ref, Wu_ref, bu_ref, o_ref, y_ref = refs
        elif act:
            G_ref, W_ref, b_ref, g_ref, bt_ref, o_ref, y_ref = refs
        else:
            G_ref, W_ref, b_ref, o_ref, y_ref = refs
        y_ref[...] = _dot_t(G_ref[...], W_ref[...]) + b_ref[...]
        if act:
            mask = (lax.broadcasted_iota(jnp.int32, (R, 1), 0) < n
                    ).astype(jnp.float32)
            mu = jnp.sum(y_ref[...] * mask, 0, keepdims=True) * (1.0 / n)
            yc = y_ref[...] - mu
            var = jnp.sum(yc * yc * mask, 0, keepdims=True) * (1.0 / n)
            y_ref[...] = jnp.tanh(
                (y_ref[...] - mu) * lax.rsqrt(var + 1e-5) * g_ref[...]
                + bt_ref[...])
        if up:
            o_ref[...] = _dot_t(y_ref[...], Wu_ref[...]) + bu_ref[...]
        else:
            o_ref[...] = y_ref[...]

    ins = [G, Wm, b.reshape(1, fo)]
    if act:
        ins += [g.reshape(1, fo), bt.reshape(1, fo)]
    if up:
        ins += [Wu, bu.reshape(1, fu)]
    return pl.pallas_call(
        body,
        out_shape=jax.ShapeDtypeStruct((R, fu), jnp.float32),
        scratch_shapes=[pltpu.VMEM((R, fo), jnp.float32)],
    )(*ins)


def _tc_matmul(A, M):
    R = int(A.shape[0])
    f = int(M.shape[1])

    def body(A_ref, M_ref, o_ref):
        o_ref[...] = lax.dot_general(
            A_ref[...], M_ref[...], (((1,), (0,)), ((), ())),
            preferred_element_type=jnp.float32,
            precision=lax.Precision.DEFAULT,
        )

    return pl.pallas_call(
        body,
        out_shape=jax.ShapeDtypeStruct((R, f), jnp.float32),
    )(A, M)


def _pool_mat(f):
    S = np.zeros((7 * f, f), np.float32)
    S[np.arange(7 * f), np.repeat(np.arange(f), 7)] = 1.0 / 7.0
    return jnp.asarray(S)


def _pair_mat(f):
    P = np.zeros((2 * f, f), np.float32)
    P[np.arange(2 * f), np.repeat(np.arange(f), 2)] = 0.5
    return jnp.asarray(P)


def kernel(x, params, neigh_10242, neigh_2562, neigh_642, neigh_162,
           up_top_642, up_down_642, up_top_2562, up_down_2562,
           up_top_10242, up_down_10242):
    p = params

    x16 = jnp.pad(x, ((0, 0), (0, 12)))
    W11 = p['c1_1_W'].reshape(32, 7, 4)
    W11 = jnp.pad(W11, ((0, 0), (0, 0), (0, 12))).reshape(32, 112)

    i1 = _pad_rows_idx(neigh_10242, _BLK)
    i2 = _pad_rows_idx(neigh_2562, _BLK)
    i3 = _pad_rows_idx(neigh_642, _BLK)
    i4 = _pad_rows_idx(neigh_162, _BLK)

    def conv(h, idx, n, name, Wm=None, up=None):
        D = int(h.shape[1])
        G = _sc_gather(h, idx).reshape(-1, 7 * D)
        Wm = p[name + '_W'] if Wm is None else Wm
        if name == 'c10':
            return _tc_conv(G, Wm, p[name + '_b'], None, None, n)
        Wu, bu = (p[up + '_W'], p[up + '_b']) if up else (None, None)
        return _tc_conv(G, Wm, p[name + '_b'], p[name + '_g'],
                        p[name + '_bt'], n, Wu, bu)

    def pool(h, neigh_full, num):
        D = int(h.shape[1])
        idx = _pad_rows_idx(neigh_full[:num * 7], _BLK)
        G = _sc_gather(h, idx).reshape(-1, 7 * D)
        return _tc_matmul(G, _pool_mat(D))

    def up_finish(y, down, raw, num, fo, skip):
        tab = y.reshape(-1, fo)
        x1 = y[:raw, :fo]
        Z = _sc_gather(tab, _pad_rows_idx(down, _BLK)).reshape(-1, 2 * fo)
        x2 = _tc_matmul(Z, _pair_mat(fo))[:num - raw]
        return jnp.concatenate([jnp.concatenate([x1, x2], 0), skip[:num]], 1)

    h = conv(x16, i1, 10242, 'c1_1', Wm=W11)
    x1s = conv(h, i1, 10242, 'c1_2')
    h = pool(x1s, neigh_10242, 2562)
    h = conv(h, i2, 2562, 'c2_1')
    x2s = conv(h, i2, 2562, 'c2_2')
    h = pool(x2s, neigh_2562, 642)
    h = conv(h, i3, 642, 'c3_1')
    x3s = conv(h, i3, 642, 'c3_2')
    h = pool(x3s, neigh_642, 162)
    h = conv(h, i4, 162, 'c4_1')
    y1 = conv(h, i4, 162, 'c4_2', up='u1')
    h = up_finish(y1, up_down_642, 162, 642, 128, x3s)
    h = conv(h, i3, 642, 'c7_1')
    y2 = conv(h, i3, 642, 'c7_2', up='u2')
    h = up_finish(y2, up_down_2562, 642, 2562, 64, x2s)
    h = conv(h, i2, 2562, 'c8_1')
    y3 = conv(h, i2, 2562, 'c8_2', up='u3')
    h = up_finish(y3, up_down_10242, 2562, 10242, 32, x1s)
    h = conv(h, i1, 10242, 'c9_1')
    h = conv(h, i1, 10242, 'c9_2')
    return conv(h, i1, 10242, 'c10')[:10242]


# speedup vs baseline: 4.2400x; 1.0747x over previous
import functools

import numpy as np
import jax
import jax.numpy as jnp
from jax import lax
from jax.experimental import pallas as pl
from jax.experimental.pallas import tpu as pltpu
from jax.experimental.pallas import tpu_sc as plsc

def _pad_rows_idx(idx, mult):
    r = (-idx.shape[0]) % mult
    return jnp.pad(idx, (0, r)) if r else idx


_W = 128
_BLK = 7 * _W


def _sc_gather(table, idx):
    B = int(idx.shape[0])
    V = int(table.shape[0])
    D = int(table.shape[1])
    mesh = plsc.VectorSubcoreMesh(core_axis_name="c", subcore_axis_name="s")

    def body(table_hbm, idx_hbm, out_hbm, tab_sh):
        @pl.when(lax.axis_index("s") == 0)
        def _():
            pltpu.sync_copy(table_hbm, tab_sh)
        plsc.subcore_barrier()

        def inner(i_vmem, o_vmem):
            pltpu.sync_copy(tab_sh.at[i_vmem.at[0]], o_vmem)

        pltpu.emit_pipeline(
            inner,
            grid=(B // _W,),
            in_specs=[pl.BlockSpec((1, _W), index_map=lambda i: (0, i))],
            out_specs=[pl.BlockSpec((_W, D), index_map=lambda i: (i, 0))],
            core_axis_name=("c", "s"),
            dimension_semantics=(pltpu.PARALLEL,),
        )(idx_hbm, out_hbm)

    k = pl.kernel(
        body,
        out_type=jax.ShapeDtypeStruct((B, D), jnp.float32),
        mesh=mesh,
        scratch_types=[pltpu.VMEM_SHARED((V, D), jnp.float32)],
        compiler_params=pltpu.CompilerParams(use_tc_tiling_on_sc=False),
    )
    return k(table, idx.reshape(1, B))


def _dot_t(a, w):
    return lax.dot_general(
        a, w, (((1,), (1,)), ((), ())),
        preferred_element_type=jnp.float32,
        precision=lax.Precision.DEFAULT,
    )


def _tc_conv(G, Wm, b, g, bt, n, Wu=None, bu=None):
    R = int(G.shape[0])
    fo = int(Wm.shape[0])
    act = g is not None
    up = Wu is not None
    fu = int(Wu.shape[0]) if up else fo

    def body(*refs):
        if act and up:
            G_ref, W_ref, b_ref, g_ref, bt_ref, Wu_ref, bu_ref, o_ref, y_ref = refs
        elif act:
            G_ref, W_ref, b_ref, g_ref, bt_ref, o_ref, y_ref = refs
        else:
            G_ref, W_ref, b_ref, o_ref, y_ref = refs
        y_ref[...] = _dot_t(G_ref[...], W_ref[...]) + b_ref[...]
        if act:
            mask = (lax.broadcasted_iota(jnp.int32, (R, 1), 0) < n
                    ).astype(jnp.float32)
            mu = jnp.sum(y_ref[...] * mask, 0, keepdims=True) * (1.0 / n)
            yc = y_ref[...] - mu
            var = jnp.sum(yc * yc * mask, 0, keepdims=True) * (1.0 / n)
            y_ref[...] = jnp.tanh(
                (y_ref[...] - mu) * lax.rsqrt(var + 1e-5) * g_ref[...]
                + bt_ref[...])
        if up:
            o_ref[...] = _dot_t(y_ref[...], Wu_ref[...]) + bu_ref[...]
        else:
            o_ref[...] = y_ref[...]

    ins = [G, Wm, b.reshape(1, fo)]
    if act:
        ins += [g.reshape(1, fo), bt.reshape(1, fo)]
    if up:
        ins += [Wu, bu.reshape(1, fu)]
    return pl.pallas_call(
        body,
        out_shape=jax.ShapeDtypeStruct((R, fu), jnp.float32),
        scratch_shapes=[pltpu.VMEM((R, fo), jnp.float32)],
    )(*ins)


def _tc_matmul(A, M):
    R = int(A.shape[0])
    f = int(M.shape[1])

    def body(A_ref, M_ref, o_ref):
        o_ref[...] = lax.dot_general(
            A_ref[...], M_ref[...], (((1,), (0,)), ((), ())),
            preferred_element_type=jnp.float32,
            precision=lax.Precision.DEFAULT,
        )

    return pl.pallas_call(
        body,
        out_shape=jax.ShapeDtypeStruct((R, f), jnp.float32),
    )(A, M)


def _mix_mats(nout, g, D, npad):
    Rs = np.zeros((g, npad, nout * g), np.float32)
    Cs = np.zeros((g, D, D), np.float32)
    for t in range(g):
        Rs[t, np.arange(nout), np.arange(nout) * g + t] = 1.0
    for c in range(D):
        for j in range(g):
            e = g * c + j
            Cs[e // D, e % D, c] += 1.0 / g
    return (jnp.asarray(Rs.reshape(g * npad, nout * g)),
            jnp.asarray(Cs.reshape(g * D, D)))


def _tc_conv_oh(X, idx7, n, Wm, b, g, bt, Wu=None, bu=None):
    Rx, fi = int(X.shape[0]), int(X.shape[1])
    Ro = int(idx7.shape[1])
    fo = int(Wm.shape[0])
    W7 = jnp.transpose(Wm.reshape(fo, 7, fi), (1, 0, 2))
    act = g is not None
    up = Wu is not None
    fu = int(Wu.shape[0]) if up else fo

    def body(*refs):
        if act and up:
            X_ref, i_ref, W_ref, b_ref, g_ref, bt_ref, Wu_ref, bu_ref, \
                o_ref, y_ref = refs
        elif act:
            X_ref, i_ref, W_ref, b_ref, g_ref, bt_ref, o_ref, y_ref = refs
        else:
            X_ref, i_ref, W_ref, b_ref, o_ref, y_ref = refs
        for k in range(7):
            Hk = _dot_t(X_ref[...], W_ref[k])
            oh = (i_ref[k] == lax.broadcasted_iota(jnp.int32, (Ro, Rx), 1)
                  ).astype(jnp.float32)
            term = lax.dot_general(
                oh, Hk, (((1,), (0,)), ((), ())),
                preferred_element_type=jnp.float32,
                precision=lax.Precision.DEFAULT)
            if k == 0:
                y_ref[...] = term + b_ref[...]
            else:
                y_ref[...] = y_ref[...] + term
        if act:
            mu = jnp.mean(y_ref[...], 0, keepdims=True)
            yc = y_ref[...] - mu
            var = jnp.mean(yc * yc, 0, keepdims=True)
            y_ref[...] = jnp.tanh(
                (y_ref[...] - mu) * lax.rsqrt(var + 1e-5) * g_ref[...]
                + bt_ref[...])
        if up:
            o_ref[...] = _dot_t(y_ref[...], Wu_ref[...]) + bu_ref[...]
        else:
            o_ref[...] = y_ref[...]

    ins = [X, idx7, W7, b.reshape(1, fo)]
    if act:
        ins += [g.reshape(1, fo), bt.reshape(1, fo)]
    if up:
        ins += [Wu, bu.reshape(1, fu)]
    return pl.pallas_call(
        body,
        out_shape=jax.ShapeDtypeStruct((Ro, fu), jnp.float32),
        scratch_shapes=[pltpu.VMEM((Ro, fo), jnp.float32)],
    )(*ins)


def _dot(a, m):
    return lax.dot_general(
        a, m, (((1,), (0,)), ((), ())),
        preferred_element_type=jnp.float32,
        precision=lax.Precision.DEFAULT)


def _tc_gather_mix(X, idx, nout, gsize, npad):
    Rx, D = int(X.shape[0]), int(X.shape[1])
    nidx = int(idx.shape[0])
    Rs, Cs = _mix_mats(nout, gsize, D, npad)

    def body(X_ref, i_ref, R_ref, C_ref, o_ref, fl_ref):
        oh = (i_ref[...] == lax.broadcasted_iota(jnp.int32, (nidx, Rx), 1)
              ).astype(jnp.float32)
        fl_ref[...] = _dot(oh, X_ref[...])
        for t in range(gsize):
            term = _dot(_dot(R_ref[pl.ds(t * npad, npad)], fl_ref[...]),
                        C_ref[pl.ds(t * D, D)])
            o_ref[...] = term if t == 0 else o_ref[...] + term

    return pl.pallas_call(
        body,
        out_shape=jax.ShapeDtypeStruct((npad, D), jnp.float32),
        scratch_shapes=[pltpu.VMEM((nidx, D), jnp.float32)],
    )(X, idx, Rs, Cs)


def _tc_up_small(yflat, down, raw, num, fo):
    nf = int(yflat.shape[0])
    nd = int(down.shape[0])
    m = num - raw
    Rtop = np.zeros((num, nf), np.float32)
    Rtop[np.arange(raw), np.arange(raw) * 7] = 1.0
    RRcat = np.zeros((num, 2 * nd), np.float32)
    Cs = np.zeros((2, fo, fo), np.float32)
    for t in range(2):
        RRcat[raw + np.arange(m), t * nd + np.arange(m) * 2 + t] = 1.0
    for c in range(fo):
        for j in range(2):
            e = 2 * c + j
            Cs[e // fo, e % fo, c] += 0.5
    Rtop, RRcat, Cs = (jnp.asarray(Rtop), jnp.asarray(RRcat),
                       jnp.asarray(Cs.reshape(2 * fo, fo)))

    def body(y_ref, d_ref, Rt_ref, RR_ref, C_ref, o_ref, z_ref, z2_ref):
        oh = (d_ref[...] == lax.broadcasted_iota(jnp.int32, (nd, nf), 1)
              ).astype(jnp.float32)
        z_ref[...] = _dot(oh, y_ref[...])
        for t in range(2):
            z2_ref[pl.ds(t * nd, nd)] = _dot(z_ref[...],
                                             C_ref[pl.ds(t * fo, fo)])
        o_ref[...] = _dot(Rt_ref[...], y_ref[...]) + _dot(RR_ref[...],
                                                          z2_ref[...])

    return pl.pallas_call(
        body,
        out_shape=jax.ShapeDtypeStruct((num, fo), jnp.float32),
        scratch_shapes=[pltpu.VMEM((nd, fo), jnp.float32),
                        pltpu.VMEM((2 * nd, fo), jnp.float32)],
    )(yflat, down, Rtop, RRcat, Cs)


def _pool_mat(f):
    S = np.zeros((7 * f, f), np.float32)
    S[np.arange(7 * f), np.repeat(np.arange(f), 7)] = 1.0 / 7.0
    return jnp.asarray(S)


def _pair_mat(f):
    P = np.zeros((2 * f, f), np.float32)
    P[np.arange(2 * f), np.repeat(np.arange(f), 2)] = 0.5
    return jnp.asarray(P)


def kernel(x, params, neigh_10242, neigh_2562, neigh_642, neigh_162,
           up_top_642, up_down_642, up_top_2562, up_down_2562,
           up_top_10242, up_down_10242):
    p = params

    x16 = jnp.pad(x, ((0, 0), (0, 12)))
    W11 = p['c1_1_W'].reshape(32, 7, 4)
    W11 = jnp.pad(W11, ((0, 0), (0, 0), (0, 12))).reshape(32, 112)

    i1 = _pad_rows_idx(neigh_10242, _BLK)
    i2 = _pad_rows_idx(neigh_2562, _BLK)
    i3m = neigh_642.reshape(642, 7).T[:, :, None]
    i4m = neigh_162.reshape(162, 7).T[:, :, None]

    def conv(h, idx, n, name, Wm=None, up=None):
        D = int(h.shape[1])
        G = _sc_gather(h, idx).reshape(-1, 7 * D)
        Wm = p[name + '_W'] if Wm is None else Wm
        if name == 'c10':
            return _tc_conv(G, Wm, p[name + '_b'], None, None, n)
        Wu, bu = (p[up + '_W'], p[up + '_b']) if up else (None, None)
        return _tc_conv(G, Wm, p[name + '_b'], p[name + '_g'],
                        p[name + '_bt'], n, Wu, bu)

    def pool(h, neigh_full, num):
        D = int(h.shape[1])
        idx = _pad_rows_idx(neigh_full[:num * 7], _BLK)
        G = _sc_gather(h, idx).reshape(-1, 7 * D)
        return _tc_matmul(G, _pool_mat(D))

    def up_finish(y, down, raw, num, fo, skip):
        tab = y.reshape(-1, fo)
        x1 = y[:raw, :fo]
        Z = _sc_gather(tab, _pad_rows_idx(down, _BLK)).reshape(-1, 2 * fo)
        x2 = _tc_matmul(Z, _pair_mat(fo))[:num - raw]
        return jnp.concatenate([jnp.concatenate([x1, x2], 0), skip[:num]], 1)

    def conv_oh(h, idxm, n, name, up=None):
        Wu, bu = (p[up + '_W'], p[up + '_b']) if up else (None, None)
        return _tc_conv_oh(h, idxm, n, p[name + '_W'], p[name + '_b'],
                           p[name + '_g'], p[name + '_bt'], Wu, bu)

    h = conv(x16, i1, 10242, 'c1_1', Wm=W11)
    x1s = conv(h, i1, 10242, 'c1_2')
    h = pool(x1s, neigh_10242, 2562)
    h = conv(h, i2, 2562, 'c2_1')
    x2s = conv(h, i2, 2562, 'c2_2')
    h = pool(x2s, neigh_2562, 642)
    h = conv_oh(h, i3m, 642, 'c3_1')
    x3s = conv_oh(h, i3m, 642, 'c3_2')
    h = _tc_gather_mix(x3s, neigh_642[:162 * 7, None], 162, 7, 168)
    h = conv_oh(h, i4m, 162, 'c4_1')
    y1 = conv_oh(h, i4m, 162, 'c4_2', up='u1')
    up1 = _tc_up_small(y1.reshape(162 * 7, 128), up_down_642[:, None],
                       162, 642, 128)
    h = jnp.concatenate([up1, x3s], 1)
    h = conv_oh(h, i3m, 642, 'c7_1')
    y2 = conv_oh(h, i3m, 642, 'c7_2', up='u2')
    h = up_finish(y2, up_down_2562, 642, 2562, 64, x2s)
    h = conv(h, i2, 2562, 'c8_1')
    y3 = conv(h, i2, 2562, 'c8_2', up='u3')
    h = up_finish(y3, up_down_10242, 2562, 10242, 32, x1s)
    h = conv(h, i1, 10242, 'c9_1')
    h = conv(h, i1, 10242, 'c9_2')
    return conv(h, i1, 10242, 'c10')[:10242]
